# Initial kernel scaffold; baseline (speedup 1.0000x reference)
import jax
import jax.numpy as jnp
from jax.experimental import pallas as pl


def kernel(x, vec, edge_index, r_ij, f_ij, d_ij, node_attr, W_mix1, b_mix1, W_mix2, b_mix2, ln_g, ln_b, W_q, b_q, W_k, b_k, W_v, b_v, W_vec, W_o, b_o, W_dk, b_dk, W_dv, b_dv):
    raise NotImplementedError("write your pallas kernel here")


# speedup vs baseline: 11.5971x; 11.5971x over previous
import functools

import jax
import jax.numpy as jnp
from jax.experimental import pallas as pl
from jax.experimental.pallas import tpu as pltpu

N = 10000
E = 160000
D = 128
H = 8
DH = D // H
NRBF = 50
CUT = 5.0

BN = 2000
BE = 4000


def _silu(x):
    return x * jax.nn.sigmoid(x)


def _node_pre_body(x_ref, na_ref, vecf_ref,
                   w1a_ref, w1b_ref, b1_ref, w2_ref, b2_ref, lg_ref, lb_ref,
                   wq_ref, bq_ref, wk_ref, bk_ref, wv_ref, bv_ref, wvec_ref,
                   q_ref, kvv_ref, vec3_ref, vdot_ref):
    f32 = jnp.float32
    h = (jnp.dot(x_ref[...], w1a_ref[...], preferred_element_type=f32)
         + jnp.dot(na_ref[...], w1b_ref[...], preferred_element_type=f32)
         + b1_ref[...])
    h = _silu(h)
    h = jnp.dot(h, w2_ref[...], preferred_element_type=f32) + b2_ref[...]
    mu = h.mean(-1, keepdims=True)
    var = ((h - mu) ** 2).mean(-1, keepdims=True)
    h = (h - mu) / jnp.sqrt(var + 1e-5) * lg_ref[...] + lb_ref[...]

    q_ref[...] = jnp.dot(h, wq_ref[...], preferred_element_type=f32) + bq_ref[...]
    kvv_ref[:, 0:D] = jnp.dot(h, wk_ref[...], preferred_element_type=f32) + bk_ref[...]
    kvv_ref[:, D:4 * D] = (jnp.dot(h, wv_ref[...], preferred_element_type=f32)
                           + bv_ref[...])
    vecf = vecf_ref[...]
    kvv_ref[:, 4 * D:] = vecf

    vdot = jnp.zeros((x_ref.shape[0], D), f32)
    for c in range(3):
        vp = jnp.dot(vecf[:, c * D:(c + 1) * D], wvec_ref[...],
                     preferred_element_type=f32)
        vdot += vp[:, 0:D] * vp[:, D:2 * D]
        vec3_ref[:, c * D:(c + 1) * D] = vp[:, 2 * D:3 * D]
    vdot_ref[...] = vdot


def _node_pre(x, na, vecf, w1a, w1b, b1, w2, b2, lg, lb,
              wq, bq, wk, bk, wv, bv, wvec):
    grid = (N // BN,)
    row = lambda i: (i, 0)
    full = pl.BlockSpec((None if False else w1a.shape[0], w1a.shape[1]),
                        lambda i: (0, 0))
    def fullspec(a):
        return pl.BlockSpec(a.shape, lambda i: tuple(0 for _ in a.shape))
    in_specs = [
        pl.BlockSpec((BN, D), row),
        pl.BlockSpec((BN, D), row),
        pl.BlockSpec((BN, 3 * D), row),
    ] + [fullspec(a) for a in (w1a, w1b, b1, w2, b2, lg, lb,
                               wq, bq, wk, bk, wv, bv, wvec)]
    out_specs = [
        pl.BlockSpec((BN, D), row),
        pl.BlockSpec((BN, 7 * D), row),
        pl.BlockSpec((BN, 3 * D), row),
        pl.BlockSpec((BN, D), row),
    ]
    out_shape = [
        jax.ShapeDtypeStruct((N, D), jnp.float32),
        jax.ShapeDtypeStruct((N, 7 * D), jnp.float32),
        jax.ShapeDtypeStruct((N, 3 * D), jnp.float32),
        jax.ShapeDtypeStruct((N, D), jnp.float32),
    ]
    return pl.pallas_call(
        _node_pre_body, grid=grid, in_specs=in_specs, out_specs=out_specs,
        out_shape=out_shape,
    )(x, na, vecf, w1a, w1b, b1, w2, b2, lg, lb, wq, bq, wk, bk, wv, bv, wvec)


def _edge_dense_body(f_ref, r_ref, d_ref, wdkv_ref, bdkv_ref,
                     dkv_ref, cutd_ref):
    f32 = jnp.float32
    dkv = jnp.dot(f_ref[...], wdkv_ref[...], preferred_element_type=f32) + bdkv_ref[...]
    dkv_ref[...] = _silu(dkv)
    r = r_ref[...]
    cut = jnp.where(r < CUT, 0.5 * (jnp.cos(r * (jnp.pi / CUT)) + 1.0), 0.0)
    cutd_ref[...] = jnp.concatenate([cut, d_ref[...]], axis=1)


def _edge_dense(f, r, dvec, wdkv, bdkv):
    grid = (E // BE,)
    row = lambda i: (i, 0)
    def fullspec(a):
        return pl.BlockSpec(a.shape, lambda i: tuple(0 for _ in a.shape))
    return pl.pallas_call(
        _edge_dense_body, grid=grid,
        in_specs=[pl.BlockSpec((BE, NRBF), row), pl.BlockSpec((BE, 1), row),
                  pl.BlockSpec((BE, 3), row), fullspec(wdkv), fullspec(bdkv)],
        out_specs=[pl.BlockSpec((BE, 4 * D), row), pl.BlockSpec((BE, 4), row)],
        out_shape=[jax.ShapeDtypeStruct((E, 4 * D), jnp.float32),
                   jax.ShapeDtypeStruct((E, 4), jnp.float32)],
    )(f, r, dvec, wdkv, bdkv)


def _node_post_body(agg_ref, vec3_ref, vdot_ref, wo_ref, bo_ref,
                    dx_ref, dvec_ref):
    f32 = jnp.float32
    xa = agg_ref[:, 0:D]
    o = jnp.dot(xa, wo_ref[...], preferred_element_type=f32) + bo_ref[...]
    o1, o2, o3 = o[:, 0:D], o[:, D:2 * D], o[:, 2 * D:3 * D]
    dx_ref[...] = vdot_ref[...] * o2 + o3
    for c in range(3):
        dvec_ref[:, c * D:(c + 1) * D] = (vec3_ref[:, c * D:(c + 1) * D] * o1
                                          + agg_ref[:, D + c * D:D + (c + 1) * D])


def _node_post(agg, vec3, vdot, wo, bo):
    grid = (N // BN,)
    row = lambda i: (i, 0)
    def fullspec(a):
        return pl.BlockSpec(a.shape, lambda i: tuple(0 for _ in a.shape))
    return pl.pallas_call(
        _node_post_body, grid=grid,
        in_specs=[pl.BlockSpec((BN, 4 * D), row), pl.BlockSpec((BN, 3 * D), row),
                  pl.BlockSpec((BN, D), row), fullspec(wo), fullspec(bo)],
        out_specs=[pl.BlockSpec((BN, D), row), pl.BlockSpec((BN, 3 * D), row)],
        out_shape=[jax.ShapeDtypeStruct((N, D), jnp.float32),
                   jax.ShapeDtypeStruct((N, 3 * D), jnp.float32)],
    )(agg, vec3, vdot, wo, bo)


def _perm_v_cols(w):
    lead = w.shape[:-1]
    w = w.reshape(lead + (H, 3, DH))
    w = jnp.moveaxis(w, -2, -3)
    return w.reshape(lead + (3 * D,))


def kernel(x, vec, edge_index, r_ij, f_ij, d_ij, node_attr,
           W_mix1, b_mix1, W_mix2, b_mix2, ln_g, ln_b,
           W_q, b_q, W_k, b_k, W_v, b_v, W_vec, W_o, b_o,
           W_dk, b_dk, W_dv, b_dv):
    f32 = jnp.float32
    vecf = vec.reshape(N, 3 * D)
    w1a, w1b = W_mix1[:D], W_mix1[D:]
    wv = _perm_v_cols(W_v)
    bv = _perm_v_cols(b_v)
    wdkv = jnp.concatenate([W_dk, _perm_v_cols(W_dv)], axis=1)
    bdkv = jnp.concatenate([b_dk, _perm_v_cols(b_dv)], axis=0)

    q, kvv, vec3, vdot = _node_pre(
        x, node_attr, vecf, w1a, w1b, b_mix1, W_mix2, b_mix2, ln_g, ln_b,
        W_q, b_q, W_k, b_k, wv, bv, W_vec)
    dkv, cutd = _edge_dense(f_ij, r_ij[:, None], d_ij, wdkv, bdkv)

    src = edge_index[0]
    dst = edge_index[1]
    qd = q[dst]
    kvvs = kvv[src]
    k = kvvs[:, 0:D]
    vx, v1, v2 = kvvs[:, D:2 * D], kvvs[:, 2 * D:3 * D], kvvs[:, 3 * D:4 * D]
    dk = dkv[:, 0:D]
    dvx, dv1, dv2 = dkv[:, D:2 * D], dkv[:, 2 * D:3 * D], dkv[:, 3 * D:4 * D]
    attn = (qd * k * dk).reshape(E, H, DH).sum(-1)
    attn = _silu(attn) * cutd[:, 0:1]
    attn = jnp.repeat(attn, DH, axis=1)
    xm = vx * dvx * attn
    v1m = v1 * dv1
    v2m = v2 * dv2
    msg = [xm]
    for c in range(3):
        msg.append(kvvs[:, (4 + c) * D:(5 + c) * D] * v1m
                   + v2m * cutd[:, c + 1:c + 2])
    msg = jnp.concatenate(msg, axis=1)
    agg = jax.ops.segment_sum(msg, dst, num_segments=N)

    dx, dvec = _node_post(agg, vec3, vdot, W_o, b_o)
    return (dx, dvec.reshape(N, 3, D))
